# baseline (speedup 1.0000x reference)
import functools

import jax
import jax.numpy as jnp
from jax import lax
from jax.experimental import pallas as pl
from jax.experimental.pallas import tpu as pltpu
from jax.experimental.pallas import tpu_sc as plsc

BATCH = 16384
FACTOR = 84
NC = 2
NS = 16
NW = NC * NS
B_PER_W = BATCH // NW
G_CHUNK = 128
N_CHUNKS = B_PER_W // G_CHUNK


def _sc_kernel(user_hbm, item_hbm, eut_hbm, eit_hbm, ub_hbm, ib_hbm, b0_hbm,
               out_hbm,
               idx_u, idx_i, ut_v, it_v, ubv, ibv, outv, b0v, sem):
    wid = lax.axis_index("s") * NC + lax.axis_index("c")
    base = wid * B_PER_W

    pltpu.sync_copy(user_hbm.at[pl.ds(base, B_PER_W)], idx_u)
    pltpu.sync_copy(item_hbm.at[pl.ds(base, B_PER_W)], idx_i)
    pltpu.sync_copy(b0_hbm, b0v)

    bias_cps = []
    for g in range(N_CHUNKS):
        sl = pl.ds(g * G_CHUNK, G_CHUNK)
        bias_cps.append(pltpu.async_copy(ub_hbm.at[idx_u.at[sl]], ubv.at[sl], sem))
        bias_cps.append(pltpu.async_copy(ib_hbm.at[idx_i.at[sl]], ibv.at[sl], sem))

    def gather_f(f, carry):
        for g in range(N_CHUNKS):
            sl = pl.ds(g * G_CHUNK, G_CHUNK)
            pltpu.make_async_copy(
                eut_hbm.at[f].at[idx_u.at[sl]], ut_v.at[f, sl], sem).start()
            pltpu.make_async_copy(
                eit_hbm.at[f].at[idx_i.at[sl]], it_v.at[f, sl], sem).start()
        return carry

    lax.fori_loop(0, FACTOR, gather_f, 0)

    pltpu.make_async_copy(eut_hbm.at[:, pl.ds(0, B_PER_W)], ut_v, sem).wait()
    pltpu.make_async_copy(eit_hbm.at[:, pl.ds(0, B_PER_W)], it_v, sem).wait()
    for cp in bias_cps:
        cp.wait()

    b0 = b0v[...]

    def grp_body(g, carry):
        sl = pl.ds(g * 16, 16)

        def f_body(f, acc):
            return acc + ut_v[f, sl] * it_v[f, sl]

        acc = lax.fori_loop(0, FACTOR, f_body, jnp.zeros((16,), jnp.float32))
        outv[sl] = acc + ubv[sl] + ibv[sl] + b0
        return carry

    lax.fori_loop(0, B_PER_W // 16, grp_body, 0)

    pltpu.sync_copy(outv, out_hbm.at[pl.ds(base, B_PER_W)])


@jax.jit
def kernel(user, item, embed_user, embed_item, u_bias, i_bias, bias_):
    mesh = plsc.VectorSubcoreMesh(core_axis_name="c", subcore_axis_name="s")
    k = functools.partial(
        pl.kernel,
        mesh=mesh,
        out_type=jax.ShapeDtypeStruct((BATCH,), jnp.float32),
        compiler_params=pltpu.CompilerParams(
            needs_layout_passes=False, use_tc_tiling_on_sc=False),
        scratch_types=[
            pltpu.VMEM((B_PER_W,), jnp.int32),
            pltpu.VMEM((B_PER_W,), jnp.int32),
            pltpu.VMEM((FACTOR, B_PER_W), jnp.float32),
            pltpu.VMEM((FACTOR, B_PER_W), jnp.float32),
            pltpu.VMEM((B_PER_W,), jnp.float32),
            pltpu.VMEM((B_PER_W,), jnp.float32),
            pltpu.VMEM((B_PER_W,), jnp.float32),
            pltpu.VMEM((16,), jnp.float32),
            pltpu.SemaphoreType.DMA,
        ],
    )(_sc_kernel)
    return k(user, item, embed_user.T, embed_item.T,
             u_bias.reshape(-1), i_bias.reshape(-1),
             jnp.broadcast_to(bias_, (16,)))
